# baseline (speedup 1.0000x reference)
import jax
import jax.numpy as jnp
from jax.experimental import pallas as pl
from jax.experimental.pallas import tpu as pltpu


def _dot(a, b, dims):
    return jax.lax.dot_general(a, b, (dims, ((), ())),
                               preferred_element_type=jnp.float32)


def _gnn_kernel(x_hbm, mask_ref, w1_ref, b1_ref, wl1_ref, bl1_ref, wr1_ref,
                wl2_ref, bl2_ref, wr2_ref, out_ref, xbuf, sem):
    bsz, n, hdim = x_hbm.shape
    copies = [
        pltpu.make_async_copy(x_hbm.at[b], xbuf.at[b % 2], sem.at[b % 2])
        for b in range(bsz)
    ]
    copies[0].start()

    b1 = b1_ref[...].reshape(1, b1_ref.shape[0])
    bl1 = bl1_ref[...].reshape(1, bl1_ref.shape[0])
    bl2 = bl2_ref[...].reshape(1, 1)
    inv_cnt = 1.0 / (n - 1)

    for b in range(bsz):
        copies[b].wait()
        if b + 1 < bsz:
            copies[b + 1].start()

        h = _dot(xbuf[b % 2], w1_ref[...], (((1,), (1,)))) + b1

        nrm2 = jnp.sum(h * h, axis=1, keepdims=True)
        rn = 1.0 / jnp.maximum(jnp.sqrt(nrm2), 1e-8)
        u = h * rn
        c = nrm2 * (rn * rn)

        s1 = _dot(u, _dot(u, h, (((0,), (0,)))), (((1,), (0,))))
        agg1 = (s1 - c * h) * inv_cnt
        o1 = jnp.maximum(
            _dot(agg1, wl1_ref[...], (((1,), (1,))))
            + _dot(h, wr1_ref[...], (((1,), (1,))))
            + bl1, 0.0)

        s2 = _dot(u, _dot(u, o1, (((0,), (0,)))), (((1,), (0,))))
        agg2 = (s2 - c * o1) * inv_cnt
        z = (_dot(wl2_ref[...], agg2, (((1,), (1,))))
             + _dot(wr2_ref[...], o1, (((1,), (1,))))
             + bl2)
        out_ref[b:b + 1, :] = jax.nn.sigmoid(z) * mask_ref[b:b + 1, :]


@jax.jit
def kernel(x, mask_cls, W1, b1, Wl1, bl1, Wr1, Wl2, bl2, Wr2):
    B, N, H = x.shape
    vmem = pl.BlockSpec(memory_space=pltpu.MemorySpace.VMEM)
    return pl.pallas_call(
        _gnn_kernel,
        in_specs=[pl.BlockSpec(memory_space=pltpu.MemorySpace.HBM),
                  vmem, vmem, vmem, vmem, vmem, vmem, vmem, vmem, vmem],
        out_specs=vmem,
        out_shape=jax.ShapeDtypeStruct((B, N), jnp.float32),
        scratch_shapes=[
            pltpu.VMEM((2, N, H), jnp.float32),
            pltpu.SemaphoreType.DMA((2,)),
        ],
    )(x, mask_cls, W1, b1, Wl1, bl1, Wr1, Wl2, bl2, Wr2)


# speedup vs baseline: 1.3256x; 1.3256x over previous
import jax
import jax.numpy as jnp
from jax.experimental import pallas as pl


def _dot(a, b, dims):
    return jax.lax.dot_general(a, b, (dims, ((), ())),
                               preferred_element_type=jnp.float32)


def _gnn_kernel(x_ref, mask_ref, w1_ref, b1_ref, wl1_ref, bl1_ref, wr1_ref,
                wl2_ref, bl2_ref, wr2_ref, out_ref):
    bsz, n, hdim = x_ref.shape
    xb = x_ref[...].reshape(bsz * n, hdim)
    h_all = (_dot(xb, w1_ref[...], (((1,), (1,))))
             + b1_ref[...].reshape(1, b1_ref.shape[0]))

    bl1 = bl1_ref[...].reshape(1, bl1_ref.shape[0])
    bl2 = bl2_ref[...].reshape(1, 1)
    inv_cnt = 1.0 / (n - 1)

    for i in range(bsz):
        h = h_all[i * n:(i + 1) * n]
        g = _dot(h, h, (((1,), (1,))))
        nrm2 = jnp.sum(h * h, axis=1, keepdims=True)
        rn = 1.0 / jnp.maximum(jnp.sqrt(nrm2), 1e-8)
        a = (g * rn) * rn.reshape(1, n)
        c = nrm2 * (rn * rn)

        agg1 = (_dot(a, h, (((1,), (0,)))) - c * h) * inv_cnt
        o1 = jnp.maximum(
            _dot(agg1, wl1_ref[...], (((1,), (1,))))
            + _dot(h, wr1_ref[...], (((1,), (1,))))
            + bl1, 0.0)

        agg2 = (_dot(a, o1, (((1,), (0,)))) - c * o1) * inv_cnt
        z = (_dot(wl2_ref[...], agg2, (((1,), (1,))))
             + _dot(wr2_ref[...], o1, (((1,), (1,))))
             + bl2)
        out_ref[i:i + 1, :] = jax.nn.sigmoid(z) * mask_ref[i:i + 1, :]


@jax.jit
def kernel(x, mask_cls, W1, b1, Wl1, bl1, Wr1, Wl2, bl2, Wr2):
    B, N, H = x.shape
    return pl.pallas_call(
        _gnn_kernel,
        out_shape=jax.ShapeDtypeStruct((B, N), jnp.float32),
    )(x, mask_cls, W1, b1, Wl1, bl1, Wr1, Wl2, bl2, Wr2)


# speedup vs baseline: 1.5654x; 1.1809x over previous
import jax
import jax.numpy as jnp
from jax.experimental import pallas as pl


def _dot(a, b, dims):
    return jax.lax.dot_general(a, b, (dims, ((), ())),
                               preferred_element_type=jnp.float32)


def _gnn_kernel(x_ref, mask_ref, w1_ref, b1_ref, wl1_ref, bl1_ref, wr1_ref,
                wl2_ref, bl2_ref, wr2_ref, out_ref):
    bsz, n, hdim = x_ref.shape
    xb = x_ref[...].reshape(bsz * n, hdim)
    h_all = (_dot(xb, w1_ref[...], (((1,), (1,))))
             + b1_ref[...].reshape(1, b1_ref.shape[0]))

    bl1 = bl1_ref[...].reshape(1, bl1_ref.shape[0])
    bl2 = bl2_ref[...].reshape(1, 1)
    inv_cnt = 1.0 / (n - 1)

    nrm2 = jnp.sum(h_all * h_all, axis=1, keepdims=True)
    rn = 1.0 / jnp.maximum(jnp.sqrt(nrm2), 1e-8)
    c = nrm2 * (rn * rn)

    aa = []
    agg1_parts = []
    for i in range(bsz):
        h = h_all[i * n:(i + 1) * n]
        rni = rn[i * n:(i + 1) * n]
        g = _dot(h, h, (((1,), (1,))))
        a = (g * rni) * rni.reshape(1, n)
        aa.append(a)
        agg1_parts.append(_dot(a, h, (((1,), (0,)))))
    agg1 = (jnp.concatenate(agg1_parts, axis=0) - c * h_all) * inv_cnt

    o1_all = jnp.maximum(
        _dot(agg1, wl1_ref[...], (((1,), (1,))))
        + _dot(h_all, wr1_ref[...], (((1,), (1,))))
        + bl1, 0.0)

    for i in range(bsz):
        o1 = o1_all[i * n:(i + 1) * n]
        ci = c[i * n:(i + 1) * n]
        agg2 = (_dot(aa[i], o1, (((1,), (0,)))) - ci * o1) * inv_cnt
        z = (_dot(wl2_ref[...], agg2, (((1,), (1,))))
             + _dot(wr2_ref[...], o1, (((1,), (1,))))
             + bl2)
        out_ref[i:i + 1, :] = jax.nn.sigmoid(z) * mask_ref[i:i + 1, :]


@jax.jit
def kernel(x, mask_cls, W1, b1, Wl1, bl1, Wr1, Wl2, bl2, Wr2):
    B, N, H = x.shape
    return pl.pallas_call(
        _gnn_kernel,
        out_shape=jax.ShapeDtypeStruct((B, N), jnp.float32),
    )(x, mask_cls, W1, b1, Wl1, bl1, Wr1, Wl2, bl2, Wr2)


# speedup vs baseline: 1.5998x; 1.0219x over previous
import jax
import jax.numpy as jnp
from jax.experimental import pallas as pl


def _dot(a, b, dims):
    return jax.lax.dot_general(a, b, (dims, ((), ())),
                               preferred_element_type=jnp.float32)


def _gnn_kernel(x_ref, mask_ref, w1_ref, b1_ref, wl1_ref, bl1_ref, wr1_ref,
                wl2_ref, bl2_ref, wr2_ref, out_ref):
    bsz, n, hdim = x_ref.shape
    xb = x_ref[...].reshape(bsz * n, hdim)
    h_all = (_dot(xb, w1_ref[...], (((1,), (1,))))
             + b1_ref[...].reshape(1, b1_ref.shape[0]))

    bl1 = bl1_ref[...].reshape(1, bl1_ref.shape[0])
    bl2 = bl2_ref[...].reshape(1, 1)
    inv_cnt = 1.0 / (n - 1)

    nrm2 = jnp.sum(h_all * h_all, axis=1, keepdims=True)
    rn = 1.0 / jnp.maximum(jnp.sqrt(nrm2), 1e-8)
    c = nrm2 * (rn * rn)

    aa = []
    agg1_parts = []
    for i in range(bsz):
        h = h_all[i * n:(i + 1) * n]
        rni = rn[i * n:(i + 1) * n]
        g = _dot(h, h, (((1,), (1,))))
        a = (g * rni) * rni.reshape(1, n)
        aa.append(a)
        agg1_parts.append(_dot(a, h, (((1,), (0,)))))
    agg1 = (jnp.concatenate(agg1_parts, axis=0) - c * h_all) * inv_cnt

    o1_all = jnp.maximum(
        _dot(agg1, wl1_ref[...], (((1,), (1,))))
        + _dot(h_all, wr1_ref[...], (((1,), (1,))))
        + bl1, 0.0)

    agg2_parts = [
        _dot(aa[i], o1_all[i * n:(i + 1) * n], (((1,), (0,))))
        for i in range(bsz)
    ]
    agg2 = (jnp.concatenate(agg2_parts, axis=0) - c * o1_all) * inv_cnt
    z = (_dot(wl2_ref[...], agg2, (((1,), (1,))))
         + _dot(wr2_ref[...], o1_all, (((1,), (1,))))
         + bl2)
    out_ref[...] = jax.nn.sigmoid(z.reshape(bsz, n)) * mask_ref[...]


@jax.jit
def kernel(x, mask_cls, W1, b1, Wl1, bl1, Wr1, Wl2, bl2, Wr2):
    B, N, H = x.shape
    return pl.pallas_call(
        _gnn_kernel,
        out_shape=jax.ShapeDtypeStruct((B, N), jnp.float32),
    )(x, mask_cls, W1, b1, Wl1, bl1, Wr1, Wl2, bl2, Wr2)
